# Initial kernel scaffold; baseline (speedup 1.0000x reference)
import jax
import jax.numpy as jnp
from jax.experimental import pallas as pl


def kernel(x, W_in, W_out, k_ffwd, v_ffwd):
    raise NotImplementedError("write your pallas kernel here")


# speedup vs baseline: 2.0066x; 2.0066x over previous
import functools

import jax
import jax.numpy as jnp
import numpy as np
from jax import lax
from jax.experimental import pallas as pl

B, T, E = 1, 2048, 768
H, D = 12, 64
NE, ES, A = 8, 256, 2

BT = 256
NT = T // BT


def _rope_apply(y, cos, ssin):
    d = lax.broadcasted_iota(jnp.int32, y.shape, 1) % D
    first = d < (D // 2)
    left = jnp.concatenate([y[:, D // 2:], y[:, : D // 2]], axis=1)
    right = jnp.concatenate([y[:, -(D // 2):], y[:, : -(D // 2)]], axis=1)
    partner = jnp.where(first, left, right)
    return y * cos + partner * ssin


def _proj_kernel(x_ref, w_ref, cos_ref, ssin_ref,
                 qf_ref, qa_ref, ka_ref, va_ref, gates_ref):
    x = x_ref[...]
    h = lax.dot_general(x, w_ref[...], (((1,), (1,)), ((), ())),
                        preferred_element_type=jnp.float32)
    qf_ref[...] = h[:, :E]
    cos = cos_ref[...]
    ssin = ssin_ref[...]
    qa_ref[...] = _rope_apply(h[:, E:2 * E], cos, ssin)
    ka_ref[...] = _rope_apply(h[:, 2 * E:3 * E], cos, ssin)
    va_ref[...] = h[:, 3 * E:4 * E]
    logits = h[:, 4 * E:4 * E + NE]
    col = lax.broadcasted_iota(jnp.int32, (BT, NE), 1)
    cols = []
    for n in range(NE):
        ln = logits[:, n:n + 1]
        greater = jnp.sum((logits > ln).astype(jnp.float32), axis=1,
                          keepdims=True)
        eq_before = jnp.sum(((logits == ln) & (col < n)).astype(jnp.float32),
                            axis=1, keepdims=True)
        rank = greater + eq_before
        cols.append(jnp.where(rank < A, jax.nn.sigmoid(ln), 0.0))
    gates_ref[...] = jnp.concatenate(cols, axis=1)


def _attn_kernel(q_ref, k_ref, v_ref, o_ref):
    qi = pl.program_id(1)
    q = q_ref[0]
    k = k_ref[0]
    scores = lax.dot_general(q, k, (((1,), (1,)), ((), ())),
                             preferred_element_type=jnp.float32)
    scores = scores * (1.0 / np.sqrt(D))
    row = qi * BT + lax.broadcasted_iota(jnp.int32, scores.shape, 0)
    ccol = lax.broadcasted_iota(jnp.int32, scores.shape, 1)
    scores = jnp.where(ccol <= row, scores, -1e30)
    m = jnp.max(scores, axis=1, keepdims=True)
    p = jnp.exp(scores - m)
    p = p / jnp.sum(p, axis=1, keepdims=True)
    o_ref[0] = jnp.dot(p, v_ref[0], preferred_element_type=jnp.float32)


def _moe_out_kernel(qf_ref, gates_ref, attn_ref, kf_ref, vf_ref, w_ref,
                    o_ref):
    gates = gates_ref[...]
    expand = (lax.broadcasted_iota(jnp.int32, (NE, NE * ES), 0) ==
              lax.broadcasted_iota(jnp.int32, (NE, NE * ES), 1) // ES)
    ge = jnp.dot(gates, expand.astype(jnp.float32),
                 preferred_element_type=jnp.float32)
    ffwd_cols = []
    for h in range(H):
        qh = qf_ref[:, h * D:(h + 1) * D]
        kh = kf_ref[h]
        vh = vf_ref[h]
        s = lax.dot_general(qh, kh, (((1,), (1,)), ((), ())),
                            preferred_element_type=jnp.float32)
        a = 0.5 * s * (1.0 + lax.erf(s * np.float32(1.0 / np.sqrt(2.0))))
        ffwd_cols.append(jnp.dot(a * ge, vh,
                                 preferred_element_type=jnp.float32))
    ffwd = jnp.concatenate(ffwd_cols, axis=1)
    w = w_ref[...]
    out = lax.dot_general(attn_ref[...], w[:, :E], (((1,), (1,)), ((), ())),
                          preferred_element_type=jnp.float32)
    out += lax.dot_general(ffwd, w[:, E:], (((1,), (1,)), ((), ())),
                           preferred_element_type=jnp.float32)
    o_ref[...] = out


@jax.jit
def kernel(x, W_in, W_out, k_ffwd, v_ffwd):
    x2 = x.reshape(T, E)
    pos = np.arange(T, dtype=np.float32)
    dh = np.arange(E) % D
    inv_freq = (1.0 / (10000.0 ** (np.arange(0, D, 2, dtype=np.float32) / D)))
    ang = pos[:, None] * inv_freq[dh % (D // 2)][None, :]
    cos_t = jnp.asarray(np.cos(ang), dtype=jnp.float32)
    ssin_t = jnp.asarray(np.sin(ang) * np.where(dh < D // 2, -1.0, 1.0),
                         dtype=jnp.float32)

    qf, qa, ka, va, gates = pl.pallas_call(
        _proj_kernel,
        grid=(NT,),
        in_specs=[
            pl.BlockSpec((BT, E), lambda i: (i, 0)),
            pl.BlockSpec((4 * E + NE, E), lambda i: (0, 0)),
            pl.BlockSpec((BT, E), lambda i: (i, 0)),
            pl.BlockSpec((BT, E), lambda i: (i, 0)),
        ],
        out_specs=[
            pl.BlockSpec((BT, E), lambda i: (i, 0)),
            pl.BlockSpec((BT, E), lambda i: (i, 0)),
            pl.BlockSpec((BT, E), lambda i: (i, 0)),
            pl.BlockSpec((BT, E), lambda i: (i, 0)),
            pl.BlockSpec((BT, NE), lambda i: (i, 0)),
        ],
        out_shape=[
            jax.ShapeDtypeStruct((T, E), jnp.float32),
            jax.ShapeDtypeStruct((T, E), jnp.float32),
            jax.ShapeDtypeStruct((T, E), jnp.float32),
            jax.ShapeDtypeStruct((T, E), jnp.float32),
            jax.ShapeDtypeStruct((T, NE), jnp.float32),
        ],
    )(x2, W_in, cos_t, ssin_t)

    qa3 = qa.reshape(T, H, D).transpose(1, 0, 2)
    ka3 = ka.reshape(T, H, D).transpose(1, 0, 2)
    va3 = va.reshape(T, H, D).transpose(1, 0, 2)
    attn3 = pl.pallas_call(
        _attn_kernel,
        grid=(H, NT),
        in_specs=[
            pl.BlockSpec((1, BT, D), lambda h, qi: (h, qi, 0)),
            pl.BlockSpec((1, T, D), lambda h, qi: (h, 0, 0)),
            pl.BlockSpec((1, T, D), lambda h, qi: (h, 0, 0)),
        ],
        out_specs=pl.BlockSpec((1, BT, D), lambda h, qi: (h, qi, 0)),
        out_shape=jax.ShapeDtypeStruct((H, T, D), jnp.float32),
    )(qa3, ka3, va3)
    attn = attn3.transpose(1, 0, 2).reshape(T, E)

    kf2 = k_ffwd.reshape(H, NE * ES, D)
    vf2 = v_ffwd.reshape(H, NE * ES, D)
    out = pl.pallas_call(
        _moe_out_kernel,
        grid=(NT,),
        in_specs=[
            pl.BlockSpec((BT, E), lambda i: (i, 0)),
            pl.BlockSpec((BT, NE), lambda i: (i, 0)),
            pl.BlockSpec((BT, E), lambda i: (i, 0)),
            pl.BlockSpec((H, NE * ES, D), lambda i: (0, 0, 0)),
            pl.BlockSpec((H, NE * ES, D), lambda i: (0, 0, 0)),
            pl.BlockSpec((E, 2 * E), lambda i: (0, 0)),
        ],
        out_specs=pl.BlockSpec((BT, E), lambda i: (i, 0)),
        out_shape=jax.ShapeDtypeStruct((T, E), jnp.float32),
    )(qf, gates, attn, kf2, vf2, W_out)

    return out.reshape(B, T, E)
